# baseline (speedup 1.0000x reference)
import functools

import jax
import jax.numpy as jnp
from jax import lax
from jax.experimental import pallas as pl
from jax.experimental.pallas import tpu as pltpu
from jax.experimental.pallas import tpu_sc as plsc

T = 26
B = 1024
L = 20
ROWS = 100000
D = 64
_LANES = 16


def _make_kernel(NC, NS):
    NW = NC * NS
    BB = B // NW
    NIDX = BB * L
    CHUNK = 128
    NCHUNK = NIDX // CHUNK

    mesh = plsc.VectorSubcoreMesh(
        core_axis_name="c", subcore_axis_name="s",
        num_cores=NC, num_subcores=NS)

    @functools.partial(
        pl.kernel,
        out_type=jax.ShapeDtypeStruct((B, T * D), jnp.float32),
        mesh=mesh,
        compiler_params=pltpu.CompilerParams(use_tc_tiling_on_sc=False),
        scratch_types=[
            pltpu.VMEM((NIDX,), jnp.int32),
            pltpu.VMEM((NIDX, D), jnp.float32),
            pltpu.VMEM((BB, D), jnp.float32),
            pltpu.SemaphoreType.DMA,
        ],
    )
    def emb_kernel(idx_hbm, w_hbm, out_hbm, idx_v, rows_v, pooled_v, gsem):
        wid = lax.axis_index("s") * NC + lax.axis_index("c")
        b0 = wid * BB

        def per_table(t, carry):
            base = t * (B * L) + b0 * L
            pltpu.sync_copy(idx_hbm.at[pl.ds(base, NIDX)], idx_v)
            off = t * ROWS
            for k in range(NIDX // _LANES):
                sl = pl.ds(k * _LANES, _LANES)
                idx_v[sl] = idx_v[sl] + off
            cps = [
                pltpu.async_copy(
                    w_hbm.at[idx_v.at[pl.ds(j * CHUNK, CHUNK)]],
                    rows_v.at[pl.ds(j * CHUNK, CHUNK)], gsem)
                for j in range(NCHUNK)
            ]
            for cp in cps:
                cp.wait()

            def pool_bag(bb, c2):
                r0 = bb * L
                accs = [rows_v[r0, pl.ds(dd * _LANES, _LANES)]
                        for dd in range(D // _LANES)]
                for li in range(1, L):
                    for dd in range(D // _LANES):
                        accs[dd] = accs[dd] + rows_v[
                            r0 + li, pl.ds(dd * _LANES, _LANES)]
                for dd in range(D // _LANES):
                    pooled_v[bb, pl.ds(dd * _LANES, _LANES)] = accs[dd]
                return c2

            lax.fori_loop(0, BB, pool_bag, 0)
            pltpu.sync_copy(pooled_v,
                            out_hbm.at[pl.ds(b0, BB), pl.ds(t * D, D)])
            return carry

        lax.fori_loop(0, T, per_table, 0)

    return emb_kernel


_TCB = 16384


def _transpose_to_row_major(wT):
    def body(in_ref, out_ref):
        out_ref[...] = in_ref[...].T

    return pl.pallas_call(
        body,
        grid=(pl.cdiv(T * ROWS, _TCB),),
        in_specs=[pl.BlockSpec((D, _TCB), lambda i: (0, i))],
        out_specs=pl.BlockSpec((_TCB, D), lambda i: (i, 0)),
        out_shape=jax.ShapeDtypeStruct((T * ROWS, D), jnp.float32),
    )(wT)


def _sc_geometry():
    try:
        info = plsc.get_sparse_core_info()
        return info.num_cores, info.num_subcores
    except Exception:
        return 2, 16


def kernel(indices, offsets, weights, hash_size_cumsum):
    del offsets, hash_size_cumsum
    NC, NS = _sc_geometry()
    w = _transpose_to_row_major(weights.T)
    return _make_kernel(NC, NS)(indices, w)


# speedup vs baseline: 1.0841x; 1.0841x over previous
import functools

import jax
import jax.numpy as jnp
from jax import lax
from jax.experimental import pallas as pl
from jax.experimental.pallas import tpu as pltpu
from jax.experimental.pallas import tpu_sc as plsc

T = 26
B = 1024
L = 20
ROWS = 100000
D = 64
_LANES = 16


def _make_kernel(NC, NS):
    NW = NC * NS
    BB = B // NW
    NIDX = BB * L
    CHUNK = 128
    NCHUNK = NIDX // CHUNK
    NPAIR = T // 2

    mesh = plsc.VectorSubcoreMesh(
        core_axis_name="c", subcore_axis_name="s",
        num_cores=NC, num_subcores=NS)

    @functools.partial(
        pl.kernel,
        out_type=jax.ShapeDtypeStruct((B, T * D), jnp.float32),
        mesh=mesh,
        compiler_params=pltpu.CompilerParams(use_tc_tiling_on_sc=False),
        scratch_types=[
            pltpu.VMEM((2, NIDX), jnp.int32),
            pltpu.VMEM((2, NIDX, D), jnp.float32),
            pltpu.VMEM((2, BB, D), jnp.float32),
            pltpu.SemaphoreType.DMA,
            pltpu.SemaphoreType.DMA,
            pltpu.SemaphoreType.DMA,
            pltpu.SemaphoreType.DMA,
        ],
    )
    def emb_kernel(idx_hbm, w_hbm, out_hbm, idx_v, rows_v, pooled_v,
                   gsem0, gsem1, osem0, osem1):
        wid = lax.axis_index("s") * NC + lax.axis_index("c")
        b0 = wid * BB
        gsems = (gsem0, gsem1)
        osems = (osem0, osem1)

        def fire(t, buf):
            base = t * (B * L) + b0 * L
            pltpu.sync_copy(idx_hbm.at[pl.ds(base, NIDX)], idx_v.at[buf])
            off = t * ROWS
            for k in range(NIDX // _LANES):
                sl = pl.ds(k * _LANES, _LANES)
                idx_v[buf, sl] = idx_v[buf, sl] + off
            for j in range(NCHUNK):
                pltpu.async_copy(
                    w_hbm.at[idx_v.at[buf].at[pl.ds(j * CHUNK, CHUNK)]],
                    rows_v.at[buf].at[pl.ds(j * CHUNK, CHUNK)], gsems[buf])

        def drain(t, buf):
            for j in range(NCHUNK):
                pltpu.make_async_copy(
                    w_hbm.at[idx_v.at[buf].at[pl.ds(j * CHUNK, CHUNK)]],
                    rows_v.at[buf].at[pl.ds(j * CHUNK, CHUNK)],
                    gsems[buf]).wait()

        def pool_and_emit(t, buf, first):
            rv = rows_v.at[buf]

            def pool_bag(bb, c2):
                r0 = bb * L
                accs = [rv[r0, pl.ds(dd * _LANES, _LANES)]
                        for dd in range(D // _LANES)]
                for li in range(1, L):
                    for dd in range(D // _LANES):
                        accs[dd] = accs[dd] + rv[
                            r0 + li, pl.ds(dd * _LANES, _LANES)]
                for dd in range(D // _LANES):
                    pooled_v[buf, bb, pl.ds(dd * _LANES, _LANES)] = accs[dd]
                return c2

            lax.fori_loop(0, BB, pool_bag, 0)
            pltpu.async_copy(
                pooled_v.at[buf],
                out_hbm.at[pl.ds(b0, BB), pl.ds(t * D, D)], osems[buf])

        def wait_out(t, buf):
            pltpu.make_async_copy(
                pooled_v.at[buf],
                out_hbm.at[pl.ds(b0, BB), pl.ds(t * D, D)],
                osems[buf]).wait()

        fire(0, 0)

        def per_pair(p, carry):
            ta = 2 * p
            fire(ta + 1, 1)
            drain(ta, 0)

            @pl.when(p > 0)
            def _():
                wait_out(ta - 2, 0)

            pool_and_emit(ta, 0, p == 0)

            @pl.when(p < NPAIR - 1)
            def _():
                fire(ta + 2, 0)

            drain(ta + 1, 1)

            @pl.when(p > 0)
            def _():
                wait_out(ta - 1, 1)

            pool_and_emit(ta + 1, 1, p == 0)
            return carry

        lax.fori_loop(0, NPAIR, per_pair, 0)
        wait_out(T - 2, 0)
        wait_out(T - 1, 1)

    return emb_kernel


def _sc_geometry():
    try:
        info = plsc.get_sparse_core_info()
        return info.num_cores, info.num_subcores
    except Exception:
        return 2, 16


def kernel(indices, offsets, weights, hash_size_cumsum):
    del offsets, hash_size_cumsum
    NC, NS = _sc_geometry()
    return _make_kernel(NC, NS)(indices, weights)
